# baseline (speedup 1.0000x reference)
import functools

import jax
import jax.numpy as jnp
from jax import lax
from jax.experimental import pallas as pl
from jax.experimental.pallas import tpu as pltpu
from jax.experimental.pallas import tpu_sc as plsc

N = 10000
E = 320000
D = 128
G = 3
NC = 2
NS = 16
NW = NC * NS
EPW = E // NW
CHUNK = 50
NCHUNK = EPW // CHUNK
IB = 8
NB = NCHUNK // IB
NBUF = 4
RB = 624
ZR = 16


def _zero_fill(zbuf, nrows):
    def zrow(r, c):
        for j in range(D // 16):
            zbuf[r, pl.ds(j * 16, 16)] = jnp.zeros((16,), jnp.float32)
        return c
    lax.fori_loop(0, nrows, zrow, 0)


def _zero_acc(zbuf, acc, sid, zsem):
    for j in range(RB // ZR):
        pltpu.async_copy(zbuf, acc.at[pl.ds(sid * RB + j * ZR, ZR)], zsem)

    @pl.when(sid == NS - 1)
    def _():
        pltpu.async_copy(zbuf.at[pl.ds(0, 16)], acc.at[pl.ds(N - 16, 16)], zsem)


def _wait_zero_acc(zbuf, acc, sid, zsem):
    for j in range(RB // ZR):
        pltpu.make_async_copy(zbuf, acc.at[pl.ds(sid * RB + j * ZR, ZR)],
                              zsem).wait()

    @pl.when(sid == NS - 1)
    def _():
        pltpu.make_async_copy(zbuf.at[pl.ds(0, 16)],
                              acc.at[pl.ds(N - 16, 16)], zsem).wait()


def _drain_acc(acc, out_hbm, obase, sid):
    @pl.when(sid < NS - 1)
    def _():
        pltpu.sync_copy(acc.at[pl.ds(sid * RB, RB)],
                        out_hbm.at[pl.ds(obase + sid * RB, RB)])

    @pl.when(sid == NS - 1)
    def _():
        pltpu.sync_copy(acc.at[pl.ds((NS - 1) * RB, N - (NS - 1) * RB)],
                        out_hbm.at[pl.ds(obase + (NS - 1) * RB,
                                         N - (NS - 1) * RB)])


def _agg3_body(x_hbm, src_hbm, dst_hbm, out_hbm, sidx, didx, rows0, rows1,
               rows2, rows3, zbuf, acc, gsem0, gsem1, gsem2, gsem3,
               ssem0, ssem1, ssem2, ssem3, isem, zsem):
    cid = lax.axis_index("c")
    sid = lax.axis_index("s")
    wid = sid * NC + cid
    rows = (rows0, rows1, rows2, rows3)
    gsem = (gsem0, gsem1, gsem2, gsem3)
    ssem = (ssem0, ssem1, ssem2, ssem3)

    _zero_fill(zbuf, ZR)

    def graph(g, carry):
        pltpu.async_copy(src_hbm.at[g, wid, 0], sidx.at[0], isem)
        pltpu.async_copy(dst_hbm.at[g, wid, 0], didx.at[0], isem)

        _zero_acc(zbuf, acc, sid, zsem)

        pltpu.make_async_copy(src_hbm.at[g, wid, 0], sidx.at[0], isem).wait()
        pltpu.make_async_copy(dst_hbm.at[g, wid, 0], didx.at[0], isem).wait()
        pltpu.async_copy(src_hbm.at[g, wid, 1], sidx.at[1], isem)
        pltpu.async_copy(dst_hbm.at[g, wid, 1], didx.at[1], isem)

        for b in range(NBUF):
            pltpu.async_copy(x_hbm.at[sidx.at[0, b]], rows[b], gsem[b])

        _wait_zero_acc(zbuf, acc, sid, zsem)
        plsc.subcore_barrier()

        def outer(i, c):
            p = i % 2
            q = 1 - p

            @pl.when((i >= 1) & (i + 1 < NB))
            def _():
                pltpu.async_copy(src_hbm.at[g, wid, i + 1], sidx.at[q], isem)
                pltpu.async_copy(dst_hbm.at[g, wid, i + 1], didx.at[q], isem)

            for l in range(IB):
                b = l % NBUF
                pltpu.make_async_copy(x_hbm.at[sidx.at[p, l]], rows[b],
                                      gsem[b]).wait()
                pltpu.async_copy(rows[b], acc.at[didx.at[p, l]], ssem[b],
                                 add=True)
                if l < IB - NBUF:
                    pltpu.make_async_copy(rows[b], acc.at[didx.at[p, l]],
                                          ssem[b]).wait()
                    pltpu.async_copy(x_hbm.at[sidx.at[p, l + NBUF]], rows[b],
                                     gsem[b])
                else:
                    @pl.when(i + 1 < NB)
                    def _(l=l, b=b):
                        if l == IB - NBUF:
                            pltpu.make_async_copy(src_hbm.at[g, wid, i + 1],
                                                  sidx.at[q], isem).wait()
                            pltpu.make_async_copy(dst_hbm.at[g, wid, i + 1],
                                                  didx.at[q], isem).wait()
                        pltpu.make_async_copy(rows[b], acc.at[didx.at[p, l]],
                                              ssem[b]).wait()
                        pltpu.async_copy(x_hbm.at[sidx.at[q, l - (IB - NBUF)]],
                                         rows[b], gsem[b])
            return c
        lax.fori_loop(0, NB, outer, 0)

        for l in range(IB - NBUF, IB):
            b = l % NBUF
            pltpu.make_async_copy(rows[b], acc.at[didx.at[0, l]],
                                  ssem[b]).wait()
        plsc.subcore_barrier()

        _drain_acc(acc, out_hbm, g * 2 * N + cid * N, sid)
        plsc.subcore_barrier()
        return carry
    lax.fori_loop(0, G, graph, 0)


def _cnt3_body(dst_hbm, out_hbm, didx, ones, zbuf, acc, ssem0, ssem1, isem,
               zsem):
    cid = lax.axis_index("c")
    sid = lax.axis_index("s")
    wid = sid * NC + cid
    ssem = (ssem0, ssem1)

    def orow(r, c):
        for j in range(D // 16):
            ones[r, pl.ds(j * 16, 16)] = jnp.ones((16,), jnp.float32)
        return c
    lax.fori_loop(0, CHUNK, orow, 0)
    _zero_fill(zbuf, ZR)

    def graph(g, carry):
        pltpu.async_copy(dst_hbm.at[g, pl.ds(wid * NCHUNK, NCHUNK)], didx,
                         isem)
        _zero_acc(zbuf, acc, sid, zsem)
        pltpu.make_async_copy(dst_hbm.at[g, pl.ds(wid * NCHUNK, NCHUNK)],
                              didx, isem).wait()
        _wait_zero_acc(zbuf, acc, sid, zsem)
        plsc.subcore_barrier()

        pltpu.async_copy(ones, acc.at[didx.at[0]], ssem0, add=True)
        pltpu.async_copy(ones, acc.at[didx.at[1]], ssem1, add=True)

        def pair(j2, c):
            for b in range(2):
                jj = j2 * 2 + b
                pltpu.make_async_copy(ones, acc.at[didx.at[jj - 2]],
                                      ssem[b]).wait()
                pltpu.async_copy(ones, acc.at[didx.at[jj]], ssem[b], add=True)
            return c
        lax.fori_loop(1, NCHUNK // 2, pair, 0)

        pltpu.make_async_copy(ones, acc.at[didx.at[NCHUNK - 2]], ssem0).wait()
        pltpu.make_async_copy(ones, acc.at[didx.at[NCHUNK - 1]], ssem1).wait()
        plsc.subcore_barrier()

        _drain_acc(acc, out_hbm, g * 2 * N + cid * N, sid)
        plsc.subcore_barrier()
        return carry
    lax.fori_loop(0, G, graph, 0)


@functools.cache
def _make_agg3():
    return pl.kernel(
        _agg3_body,
        out_type=jax.ShapeDtypeStruct((G * 2 * N, D), jnp.float32),
        mesh=plsc.VectorSubcoreMesh(core_axis_name="c", subcore_axis_name="s"),
        scratch_types=[
            pltpu.VMEM((2, IB, CHUNK), jnp.int32),
            pltpu.VMEM((2, IB, CHUNK), jnp.int32),
            pltpu.VMEM((CHUNK, D), jnp.float32),
            pltpu.VMEM((CHUNK, D), jnp.float32),
            pltpu.VMEM((CHUNK, D), jnp.float32),
            pltpu.VMEM((CHUNK, D), jnp.float32),
            pltpu.VMEM((ZR, D), jnp.float32),
            pltpu.VMEM_SHARED((N, D), jnp.float32),
            pltpu.SemaphoreType.DMA,
            pltpu.SemaphoreType.DMA,
            pltpu.SemaphoreType.DMA,
            pltpu.SemaphoreType.DMA,
            pltpu.SemaphoreType.DMA,
            pltpu.SemaphoreType.DMA,
            pltpu.SemaphoreType.DMA,
            pltpu.SemaphoreType.DMA,
            pltpu.SemaphoreType.DMA,
            pltpu.SemaphoreType.DMA,
        ],
        name="sage_agg3",
    )


@functools.cache
def _make_cnt3():
    return pl.kernel(
        _cnt3_body,
        out_type=jax.ShapeDtypeStruct((G * 2 * N, D), jnp.float32),
        mesh=plsc.VectorSubcoreMesh(core_axis_name="c", subcore_axis_name="s"),
        scratch_types=[
            pltpu.VMEM((NCHUNK, CHUNK), jnp.int32),
            pltpu.VMEM((CHUNK, D), jnp.float32),
            pltpu.VMEM((ZR, D), jnp.float32),
            pltpu.VMEM_SHARED((N, D), jnp.float32),
            pltpu.SemaphoreType.DMA,
            pltpu.SemaphoreType.DMA,
            pltpu.SemaphoreType.DMA,
            pltpu.SemaphoreType.DMA,
        ],
        name="sage_cnt3",
    )


ROWS_BLK = 1000


def _dense3_body(x_ref, p0_ref, p1_ref, c0_ref, c1_ref, wst_ref, wnt_ref,
                 b_ref, o_ref):
    ssum = p0_ref[...] + p1_ref[...]
    cnt = c0_ref[:, 0:1] + c1_ref[:, 0:1]
    mean = ssum * (1.0 / jnp.maximum(cnt, 1.0))
    h = jnp.dot(x_ref[...], wst_ref[0], preferred_element_type=jnp.float32)
    h = h + jnp.dot(mean, wnt_ref[0], preferred_element_type=jnp.float32)
    h = h + b_ref[0]
    h = jnp.maximum(h, 0.0)
    nrm = jnp.sqrt(jnp.sum(h * h, axis=1, keepdims=True))
    o_ref[...] = h / jnp.maximum(nrm, 1e-12)


@functools.cache
def _make_dense3():
    nblk = N // ROWS_BLK
    pb = 2 * N // ROWS_BLK
    return pl.pallas_call(
        _dense3_body,
        grid=(G, nblk),
        in_specs=[
            pl.BlockSpec((ROWS_BLK, D), lambda g, i: (g * (N // ROWS_BLK) + i, 0)),
            pl.BlockSpec((ROWS_BLK, D), lambda g, i: (g * pb + i, 0)),
            pl.BlockSpec((ROWS_BLK, D),
                         lambda g, i: (g * pb + N // ROWS_BLK + i, 0)),
            pl.BlockSpec((ROWS_BLK, 16), lambda g, i: (g * pb + i, 0)),
            pl.BlockSpec((ROWS_BLK, 16),
                         lambda g, i: (g * pb + N // ROWS_BLK + i, 0)),
            pl.BlockSpec((1, D, D), lambda g, i: (g, 0, 0)),
            pl.BlockSpec((1, D, D), lambda g, i: (g, 0, 0)),
            pl.BlockSpec((1, 1, D), lambda g, i: (g, 0, 0)),
        ],
        out_specs=pl.BlockSpec((ROWS_BLK, D),
                               lambda g, i: (g * (N // ROWS_BLK) + i, 0)),
        out_shape=jax.ShapeDtypeStruct((G * N, D), jnp.float32),
        name="sage_dense3",
    )


def _norm_body(x_ref, o_ref):
    h = x_ref[...]
    nrm = jnp.sqrt(jnp.sum(h * h, axis=1, keepdims=True))
    o_ref[...] = h / jnp.maximum(nrm, 1e-12)


@functools.cache
def _make_norm3():
    return pl.pallas_call(
        _norm_body,
        grid=(G * N // ROWS_BLK,),
        in_specs=[pl.BlockSpec((ROWS_BLK, D), lambda i: (i, 0))],
        out_specs=pl.BlockSpec((ROWS_BLK, D), lambda i: (i, 0)),
        out_shape=jax.ShapeDtypeStruct((G * N, D), jnp.float32),
        name="row_norm3",
    )


def kernel(x0, x1, x2, edge_index0, edge_index1, edge_index2,
           W_self0, W_neigh0, b0,
           W_self1, W_neigh1, b1,
           W_self2, W_neigh2, b2):
    agg3 = _make_agg3()
    cnt3 = _make_cnt3()
    dense3 = _make_dense3()
    norm3 = _make_norm3()

    eis = [edge_index0, edge_index1, edge_index2]
    src3 = jnp.stack([eis[g][0] + g * N for g in range(G)]) \
              .reshape(G, NW, NB, IB, CHUNK)
    dst3 = jnp.stack([eis[g][1] for g in range(G)]) \
              .reshape(G, NW, NB, IB, CHUNK)
    dst3c = jnp.stack([eis[g][1] for g in range(G)]) \
               .reshape(G, E // CHUNK, CHUNK)
    wst = jnp.stack([W_self0.T, W_self1.T, W_self2.T])
    wnt = jnp.stack([W_neigh0.T, W_neigh1.T, W_neigh2.T])
    bia = jnp.stack([b0, b1, b2]).reshape(G, 1, D)

    xall = norm3(jnp.concatenate([x0, x1, x2], axis=0))
    cnt_all = cnt3(dst3c)[:, :16]

    for _ in range(3):
        aggs = agg3(xall, src3, dst3)
        xall = dense3(xall, aggs, aggs, cnt_all, cnt_all, wst, wnt, bia)
    return (xall[:N], xall[N:2 * N], xall[2 * N:])


# speedup vs baseline: 1.1227x; 1.1227x over previous
import functools

import jax
import jax.numpy as jnp
from jax import lax
from jax.experimental import pallas as pl
from jax.experimental.pallas import tpu as pltpu
from jax.experimental.pallas import tpu_sc as plsc

N = 10000
E = 320000
D = 128
NC = 2
NS = 16
NW = NC * NS
EPW = E // NW
CHUNK = 50
NCHUNK = EPW // CHUNK
IB = 8
NB = NCHUNK // IB
NBUF = 4
RB = 624
ZR = 16


def _zero_fill(zbuf, nrows):
    def zrow(r, c):
        for j in range(D // 16):
            zbuf[r, pl.ds(j * 16, 16)] = jnp.zeros((16,), jnp.float32)
        return c
    lax.fori_loop(0, nrows, zrow, 0)


def _zero_acc(zbuf, acc, sid, zsem):
    for j in range(RB // ZR):
        pltpu.async_copy(zbuf, acc.at[pl.ds(sid * RB + j * ZR, ZR)], zsem)

    @pl.when(sid == NS - 1)
    def _():
        pltpu.async_copy(zbuf.at[pl.ds(0, 16)], acc.at[pl.ds(N - 16, 16)], zsem)


def _wait_zero_acc(zbuf, acc, sid, zsem):
    for j in range(RB // ZR):
        pltpu.make_async_copy(zbuf, acc.at[pl.ds(sid * RB + j * ZR, ZR)],
                              zsem).wait()

    @pl.when(sid == NS - 1)
    def _():
        pltpu.make_async_copy(zbuf.at[pl.ds(0, 16)],
                              acc.at[pl.ds(N - 16, 16)], zsem).wait()


def _drain_acc(acc, out_hbm, cid, sid):
    @pl.when(sid < NS - 1)
    def _():
        pltpu.sync_copy(acc.at[pl.ds(sid * RB, RB)],
                        out_hbm.at[pl.ds(cid * N + sid * RB, RB)])

    @pl.when(sid == NS - 1)
    def _():
        pltpu.sync_copy(acc.at[pl.ds((NS - 1) * RB, N - (NS - 1) * RB)],
                        out_hbm.at[pl.ds(cid * N + (NS - 1) * RB,
                                         N - (NS - 1) * RB)])


def _agg_body(x_hbm, src_hbm, dst_hbm, out_hbm, sidx, didx, rows0, rows1,
              rows2, rows3, zbuf, acc, gsem0, gsem1, gsem2, gsem3,
              ssem0, ssem1, ssem2, ssem3, isem, zsem):
    cid = lax.axis_index("c")
    sid = lax.axis_index("s")
    wid = sid * NC + cid
    rows = (rows0, rows1, rows2, rows3)
    gsem = (gsem0, gsem1, gsem2, gsem3)
    ssem = (ssem0, ssem1, ssem2, ssem3)

    pltpu.async_copy(src_hbm.at[wid, 0], sidx.at[0], isem)
    pltpu.async_copy(dst_hbm.at[wid, 0], didx.at[0], isem)

    _zero_fill(zbuf, ZR)
    _zero_acc(zbuf, acc, sid, zsem)

    pltpu.make_async_copy(src_hbm.at[wid, 0], sidx.at[0], isem).wait()
    pltpu.make_async_copy(dst_hbm.at[wid, 0], didx.at[0], isem).wait()
    pltpu.async_copy(src_hbm.at[wid, 1], sidx.at[1], isem)
    pltpu.async_copy(dst_hbm.at[wid, 1], didx.at[1], isem)

    for b in range(NBUF):
        pltpu.async_copy(x_hbm.at[sidx.at[0, b]], rows[b], gsem[b])

    _wait_zero_acc(zbuf, acc, sid, zsem)
    plsc.subcore_barrier()

    def outer(i, c):
        p = i % 2
        q = 1 - p

        @pl.when((i >= 1) & (i + 1 < NB))
        def _():
            pltpu.async_copy(src_hbm.at[wid, i + 1], sidx.at[q], isem)
            pltpu.async_copy(dst_hbm.at[wid, i + 1], didx.at[q], isem)

        for l in range(IB):
            b = l % NBUF
            pltpu.make_async_copy(x_hbm.at[sidx.at[p, l]], rows[b],
                                  gsem[b]).wait()
            pltpu.async_copy(rows[b], acc.at[didx.at[p, l]], ssem[b], add=True)
            if l < IB - NBUF:
                pltpu.make_async_copy(rows[b], acc.at[didx.at[p, l]],
                                      ssem[b]).wait()
                pltpu.async_copy(x_hbm.at[sidx.at[p, l + NBUF]], rows[b],
                                 gsem[b])
            else:
                @pl.when(i + 1 < NB)
                def _(l=l, b=b):
                    if l == IB - NBUF:
                        pltpu.make_async_copy(src_hbm.at[wid, i + 1],
                                              sidx.at[q], isem).wait()
                        pltpu.make_async_copy(dst_hbm.at[wid, i + 1],
                                              didx.at[q], isem).wait()
                    pltpu.make_async_copy(rows[b], acc.at[didx.at[p, l]],
                                          ssem[b]).wait()
                    pltpu.async_copy(x_hbm.at[sidx.at[q, l - (IB - NBUF)]],
                                     rows[b], gsem[b])
        return c
    lax.fori_loop(0, NB, outer, 0)

    for l in range(IB - NBUF, IB):
        b = l % NBUF
        pltpu.make_async_copy(rows[b], acc.at[didx.at[0, l]], ssem[b]).wait()
    plsc.subcore_barrier()

    _drain_acc(acc, out_hbm, cid, sid)


def _cnt_body(dst_hbm, out_hbm, didx, ones, zbuf, acc, ssem0, ssem1, isem,
              zsem):
    cid = lax.axis_index("c")
    sid = lax.axis_index("s")
    wid = sid * NC + cid
    ssem = (ssem0, ssem1)

    pltpu.async_copy(dst_hbm.at[pl.ds(wid * NCHUNK, NCHUNK)], didx, isem)

    def orow(r, c):
        for j in range(D // 16):
            ones[r, pl.ds(j * 16, 16)] = jnp.ones((16,), jnp.float32)
        return c
    lax.fori_loop(0, CHUNK, orow, 0)

    _zero_fill(zbuf, ZR)
    _zero_acc(zbuf, acc, sid, zsem)
    pltpu.make_async_copy(dst_hbm.at[pl.ds(wid * NCHUNK, NCHUNK)], didx,
                          isem).wait()
    _wait_zero_acc(zbuf, acc, sid, zsem)
    plsc.subcore_barrier()

    pltpu.async_copy(ones, acc.at[didx.at[0]], ssem0, add=True)
    pltpu.async_copy(ones, acc.at[didx.at[1]], ssem1, add=True)

    def pair(j2, c):
        for b in range(2):
            jj = j2 * 2 + b
            pltpu.make_async_copy(ones, acc.at[didx.at[jj - 2]],
                                  ssem[b]).wait()
            pltpu.async_copy(ones, acc.at[didx.at[jj]], ssem[b], add=True)
        return c
    lax.fori_loop(1, NCHUNK // 2, pair, 0)

    pltpu.make_async_copy(ones, acc.at[didx.at[NCHUNK - 2]], ssem0).wait()
    pltpu.make_async_copy(ones, acc.at[didx.at[NCHUNK - 1]], ssem1).wait()
    plsc.subcore_barrier()

    _drain_acc(acc, out_hbm, cid, sid)


@functools.cache
def _make_agg():
    return pl.kernel(
        _agg_body,
        out_type=jax.ShapeDtypeStruct((2 * N, D), jnp.float32),
        mesh=plsc.VectorSubcoreMesh(core_axis_name="c", subcore_axis_name="s"),
        scratch_types=[
            pltpu.VMEM((2, IB, CHUNK), jnp.int32),
            pltpu.VMEM((2, IB, CHUNK), jnp.int32),
            pltpu.VMEM((CHUNK, D), jnp.float32),
            pltpu.VMEM((CHUNK, D), jnp.float32),
            pltpu.VMEM((CHUNK, D), jnp.float32),
            pltpu.VMEM((CHUNK, D), jnp.float32),
            pltpu.VMEM((ZR, D), jnp.float32),
            pltpu.VMEM_SHARED((N, D), jnp.float32),
            pltpu.SemaphoreType.DMA,
            pltpu.SemaphoreType.DMA,
            pltpu.SemaphoreType.DMA,
            pltpu.SemaphoreType.DMA,
            pltpu.SemaphoreType.DMA,
            pltpu.SemaphoreType.DMA,
            pltpu.SemaphoreType.DMA,
            pltpu.SemaphoreType.DMA,
            pltpu.SemaphoreType.DMA,
            pltpu.SemaphoreType.DMA,
        ],
        name="sage_agg",
    )


@functools.cache
def _make_cnt():
    return pl.kernel(
        _cnt_body,
        out_type=jax.ShapeDtypeStruct((2 * N, D), jnp.float32),
        mesh=plsc.VectorSubcoreMesh(core_axis_name="c", subcore_axis_name="s"),
        scratch_types=[
            pltpu.VMEM((NCHUNK, CHUNK), jnp.int32),
            pltpu.VMEM((CHUNK, D), jnp.float32),
            pltpu.VMEM((ZR, D), jnp.float32),
            pltpu.VMEM_SHARED((N, D), jnp.float32),
            pltpu.SemaphoreType.DMA,
            pltpu.SemaphoreType.DMA,
            pltpu.SemaphoreType.DMA,
            pltpu.SemaphoreType.DMA,
        ],
        name="sage_cnt",
    )


ROWS_BLK = 1000


def _dense_body(x_ref, p0_ref, p1_ref, c0_ref, c1_ref, wst_ref, wnt_ref, b_ref,
                o_ref):
    ssum = p0_ref[...] + p1_ref[...]
    cnt = c0_ref[:, 0:1] + c1_ref[:, 0:1]
    mean = ssum * (1.0 / jnp.maximum(cnt, 1.0))
    h = jnp.dot(x_ref[...], wst_ref[...], preferred_element_type=jnp.float32)
    h = h + jnp.dot(mean, wnt_ref[...], preferred_element_type=jnp.float32)
    h = h + b_ref[...]
    h = jnp.maximum(h, 0.0)
    nrm = jnp.sqrt(jnp.sum(h * h, axis=1, keepdims=True))
    o_ref[...] = h / jnp.maximum(nrm, 1e-12)


@functools.cache
def _make_dense():
    nblk = N // ROWS_BLK
    return pl.pallas_call(
        _dense_body,
        grid=(nblk,),
        in_specs=[
            pl.BlockSpec((ROWS_BLK, D), lambda i: (i, 0)),
            pl.BlockSpec((ROWS_BLK, D), lambda i: (i, 0)),
            pl.BlockSpec((ROWS_BLK, D), lambda i: (i + nblk, 0)),
            pl.BlockSpec((ROWS_BLK, 16), lambda i: (i, 0)),
            pl.BlockSpec((ROWS_BLK, 16), lambda i: (i + nblk, 0)),
            pl.BlockSpec((D, D), lambda i: (0, 0)),
            pl.BlockSpec((D, D), lambda i: (0, 0)),
            pl.BlockSpec((1, D), lambda i: (0, 0)),
        ],
        out_specs=pl.BlockSpec((ROWS_BLK, D), lambda i: (i, 0)),
        out_shape=jax.ShapeDtypeStruct((N, D), jnp.float32),
        name="sage_dense",
    )


def _norm_body(x_ref, o_ref):
    h = x_ref[...]
    nrm = jnp.sqrt(jnp.sum(h * h, axis=1, keepdims=True))
    o_ref[...] = h / jnp.maximum(nrm, 1e-12)


@functools.cache
def _make_norm():
    return pl.pallas_call(
        _norm_body,
        grid=(N // ROWS_BLK,),
        in_specs=[pl.BlockSpec((ROWS_BLK, D), lambda i: (i, 0))],
        out_specs=pl.BlockSpec((ROWS_BLK, D), lambda i: (i, 0)),
        out_shape=jax.ShapeDtypeStruct((N, D), jnp.float32),
        name="row_norm",
    )


def kernel(x0, x1, x2, edge_index0, edge_index1, edge_index2,
           W_self0, W_neigh0, b0,
           W_self1, W_neigh1, b1,
           W_self2, W_neigh2, b2):
    agg = _make_agg()
    cntk = _make_cnt()
    dense = _make_dense()
    norm = _make_norm()

    eis = [edge_index0, edge_index1, edge_index2]
    srcs = [ei[0].reshape(NW, NB, IB, CHUNK) for ei in eis]
    dsts4 = [ei[1].reshape(NW, NB, IB, CHUNK) for ei in eis]
    dsts2 = [ei[1].reshape(E // CHUNK, CHUNK) for ei in eis]
    params = [(W_self0.T, W_neigh0.T, b0.reshape(1, D)),
              (W_self1.T, W_neigh1.T, b1.reshape(1, D)),
              (W_self2.T, W_neigh2.T, b2.reshape(1, D))]

    xs = [norm(x0), norm(x1), norm(x2)]
    cnts = [cntk(d)[:, :16] for d in dsts2]

    for _ in range(3):
        aggs = [agg(xs[k], srcs[k], dsts4[k]) for k in range(3)]
        xs = [dense(xs[k], aggs[k], aggs[k], cnts[k], cnts[k],
                    params[k][0], params[k][1], params[k][2])
              for k in range(3)]
    return (xs[0], xs[1], xs[2])
